# baseline (speedup 1.0000x reference)
import functools

import jax
import jax.numpy as jnp
from jax.experimental import pallas as pl
from jax.experimental.pallas import tpu as pltpu

_N, _P, _C, _S = 30000, 20, 64, 10
_BN = 200


def _tc_body(ts_ref, mk_ref, wta1_ref, bta1_ref, wta2_ref, bta2_ref,
             wca1_ref, bca1_ref, wca2_ref, bca2_ref, x_ref, out_ref):
    ts = ts_ref[...]
    mk = mk_ref[...]
    on = mk > 0.0

    lane_t = jax.lax.broadcasted_iota(jnp.int32, (_BN, 16), 1)
    cmax = None
    tacc = None
    for p in range(_P):
        x_p = x_ref[:, p, :]
        cmax = x_p if cmax is None else jnp.maximum(cmax, x_p)
        rm_p = jnp.max(x_p, axis=1, keepdims=True)
        sel = (ts[:, p:p + 1] == lane_t) & on[:, p:p + 1]
        term = jnp.where(sel, rm_p, 0.0)
        tacc = term if tacc is None else jnp.maximum(tacc, term)
    tmax = tacc[:, :_S]

    ta_h = jnp.maximum(
        jnp.dot(tmax, wta1_ref[...], preferred_element_type=jnp.float32)
        + bta1_ref[...], 0.0)
    ta_w = (jnp.dot(ta_h, wta2_ref[...], preferred_element_type=jnp.float32)
            + bta2_ref[...])
    ca_h = jnp.maximum(
        jnp.dot(cmax, wca1_ref[...], preferred_element_type=jnp.float32)
        + bca1_ref[...], 0.0)
    ca_w = (jnp.dot(ca_h, wca2_ref[...], preferred_element_type=jnp.float32)
            + bca2_ref[...])

    ta_g = jnp.where(ts == 0, ta_w[:, 0:1], 0.0)
    for t in range(1, _S):
        ta_g = jnp.where(ts == t, ta_w[:, t:t + 1], ta_g)

    for p in range(_P):
        z = ta_g[:, p:p + 1] * ca_w
        gate = 1.0 / (1.0 + jnp.exp(-z))
        out_ref[:, p, :] = jnp.where(on[:, p:p + 1],
                                     x_ref[:, p, :] * gate, 0.0)


@functools.partial(jax.jit, static_argnames=())
def _tca_tc(x, ts, mk, wta1t, bta1, wta2t, bta2, wca1t, bca1, wca2t, bca2):
    grid = (_N // _BN,)
    blk = lambda shape: pl.BlockSpec(shape, lambda i: (0,) * len(shape))
    return pl.pallas_call(
        _tc_body,
        grid=grid,
        in_specs=[
            pl.BlockSpec((_BN, _P), lambda i: (i, 0)),
            pl.BlockSpec((_BN, _P), lambda i: (i, 0)),
            blk(wta1t.shape), blk(bta1.shape),
            blk(wta2t.shape), blk(bta2.shape),
            blk(wca1t.shape), blk(bca1.shape),
            blk(wca2t.shape), blk(bca2.shape),
            pl.BlockSpec((_BN, _P, _C), lambda i: (i, 0, 0)),
        ],
        out_specs=pl.BlockSpec((_BN, _P, _C), lambda i: (i, 0, 0)),
        out_shape=jax.ShapeDtypeStruct((_N, _P, _C), jnp.float32),
        compiler_params=pltpu.CompilerParams(
            dimension_semantics=("parallel",)),
    )(ts, mk, wta1t, bta1, wta2t, bta2, wca1t, bca1, wca2t, bca2, x)


def kernel(x, timestep, mask, W_ta1, b_ta1, W_ta2, b_ta2,
           W_ca1, b_ca1, W_ca2, b_ca2):
    ts = timestep.astype(jnp.int32)
    mk = mask.astype(jnp.float32)
    return _tca_tc(
        x, ts, mk,
        W_ta1.T, b_ta1.reshape(1, -1),
        W_ta2.T, b_ta2.reshape(1, -1),
        W_ca1.T, b_ca1.reshape(1, -1),
        W_ca2.T, b_ca2.reshape(1, -1),
    )


# speedup vs baseline: 1.8977x; 1.8977x over previous
import functools

import jax
import jax.numpy as jnp
import numpy as np
from jax.experimental import pallas as pl
from jax.experimental.pallas import tpu as pltpu

_N, _P, _C, _S = 30000, 20, 64, 10
_BN = 512
_G = 512

_B_np = np.zeros((_P, _G), np.float32)
for _p in range(_P):
    _B_np[_p, _p * 16:(_p + 1) * 16] = 1.0
_PAT_np = np.tile(np.arange(16, dtype=np.float32), _G // 16)[None, :]
_PAD_np = np.where(np.arange(_G) < _P * 16, 0.0, -3.0e38).astype(np.float32)[None, :]
_R_np = np.zeros((_P, _P * _C), np.float32)
for _p in range(_P):
    _R_np[_p, _p * _C:(_p + 1) * _C] = 1.0
_T_np = np.zeros((_C, _P * _C), np.float32)
for _c in range(_C):
    _T_np[_c, _c::_C] = 1.0


def _dot(a, b):
    return jnp.dot(a, b, preferred_element_type=jnp.float32)


def _tc_body(ts_ref, mk_ref, wta1_ref, bta1_ref, wta2_ref, bta2_ref,
             wca1_ref, bca1_ref, wca2_ref, bca2_ref,
             bmat_ref, pat_ref, pad_ref, rmat_ref, tmat_ref,
             x3_ref, x2_ref, out_ref):
    tsf = ts_ref[...]
    mk = mk_ref[...]

    rowmax = jnp.max(x3_ref[...], axis=2)

    x2 = x2_ref[...]
    m = jnp.maximum(x2[:, :640], x2[:, 640:])
    m = jnp.maximum(
        jnp.maximum(jnp.maximum(m[:, 0:128], m[:, 128:256]),
                    jnp.maximum(m[:, 256:384], m[:, 384:512])),
        m[:, 512:640])
    cmax = jnp.maximum(m[:, :64], m[:, 64:128])

    tsx = _dot(tsf, bmat_ref[...])
    onx = _dot(mk, bmat_ref[...])
    rmx = _dot(rowmax, bmat_ref[...])
    sel = (tsx == pat_ref[...]) & (onx > 0.0)
    term = jnp.where(sel, rmx, pad_ref[...])
    m = term
    for half in (256, 128, 64, 32, 16):
        m = jnp.maximum(m[:, :half], m[:, half:2 * half])
    tmax = m[:, :_S]

    ta_h = jnp.maximum(_dot(tmax, wta1_ref[...]) + bta1_ref[...], 0.0)
    ta_w = _dot(ta_h, wta2_ref[...]) + bta2_ref[...]
    ca_h = jnp.maximum(_dot(cmax, wca1_ref[...]) + bca1_ref[...], 0.0)
    ca_w = _dot(ca_h, wca2_ref[...]) + bca2_ref[...]

    ta_g = jnp.where(tsf == 0.0, ta_w[:, 0:1], 0.0)
    for t in range(1, _S):
        ta_g = jnp.where(tsf == float(t), ta_w[:, t:t + 1], ta_g)

    ta_gx = _dot(ta_g, rmat_ref[...])
    mkx = _dot(mk, rmat_ref[...])
    ca_wx = _dot(ca_w, tmat_ref[...])
    z = ta_gx * ca_wx
    gate = 0.5 + 0.5 * jnp.tanh(0.5 * z)
    out_ref[...] = x2 * gate * mkx


@functools.partial(jax.jit, static_argnames=())
def _tca_tc(x, tsf, mk, wta1t, bta1, wta2t, bta2, wca1t, bca1, wca2t, bca2):
    x2 = x.reshape(_N, _P * _C)
    grid = ((_N + _BN - 1) // _BN,)
    blk = lambda shape: pl.BlockSpec(shape, lambda i: (0,) * len(shape))
    consts = (jnp.asarray(_B_np), jnp.asarray(_PAT_np), jnp.asarray(_PAD_np),
              jnp.asarray(_R_np), jnp.asarray(_T_np))
    out2 = pl.pallas_call(
        _tc_body,
        grid=grid,
        in_specs=[
            pl.BlockSpec((_BN, _P), lambda i: (i, 0)),
            pl.BlockSpec((_BN, _P), lambda i: (i, 0)),
            blk(wta1t.shape), blk(bta1.shape),
            blk(wta2t.shape), blk(bta2.shape),
            blk(wca1t.shape), blk(bca1.shape),
            blk(wca2t.shape), blk(bca2.shape),
            blk(consts[0].shape), blk(consts[1].shape), blk(consts[2].shape),
            blk(consts[3].shape), blk(consts[4].shape),
            pl.BlockSpec((_BN, _P, _C), lambda i: (i, 0, 0)),
            pl.BlockSpec((_BN, _P * _C), lambda i: (i, 0)),
        ],
        out_specs=pl.BlockSpec((_BN, _P * _C), lambda i: (i, 0)),
        out_shape=jax.ShapeDtypeStruct((_N, _P * _C), jnp.float32),
        compiler_params=pltpu.CompilerParams(
            dimension_semantics=("parallel",)),
    )(tsf, mk, wta1t, bta1, wta2t, bta2, wca1t, bca1, wca2t, bca2,
      *consts, x, x2)
    return out2.reshape(_N, _P, _C)


def kernel(x, timestep, mask, W_ta1, b_ta1, W_ta2, b_ta2,
           W_ca1, b_ca1, W_ca2, b_ca2):
    tsf = timestep.astype(jnp.float32)
    mk = mask.astype(jnp.float32)
    return _tca_tc(
        x, tsf, mk,
        W_ta1.T, b_ta1.reshape(1, -1),
        W_ta2.T, b_ta2.reshape(1, -1),
        W_ca1.T, b_ca1.reshape(1, -1),
        W_ca2.T, b_ca2.reshape(1, -1),
    )


# speedup vs baseline: 2.5544x; 1.3460x over previous
import functools

import jax
import jax.numpy as jnp
import numpy as np
from jax.experimental import pallas as pl
from jax.experimental.pallas import tpu as pltpu

_N, _P, _C, _S = 30000, 20, 64, 10
_BN = 512
_G = 512

_B_np = np.zeros((_P, _G), np.float32)
for _p in range(_P):
    _B_np[_p, _p * 16:(_p + 1) * 16] = 1.0
_BP_np = np.zeros((_P, _G), np.float32)
for _i in range(_P):
    _p = 2 * _i if _i < 10 else 2 * (_i - 10) + 1
    _BP_np[_i, _p * 16:(_p + 1) * 16] = 1.0
_PAT_np = np.tile(np.arange(16, dtype=np.float32), _G // 16)[None, :]
_PAD_np = np.where(np.arange(_G) < _P * 16, 0.0, -3.0e38).astype(np.float32)[None, :]
_R_np = np.zeros((_P, _P * _C), np.float32)
for _p in range(_P):
    _R_np[_p, _p * _C:(_p + 1) * _C] = 1.0
_T_np = np.zeros((_C, _P * _C), np.float32)
for _c in range(_C):
    _T_np[_c, _c::_C] = 1.0


def _dot(a, b):
    return jnp.dot(a, b, preferred_element_type=jnp.float32)


def _tc_body(ts_ref, mk_ref, wta1_ref, bta1_ref, wta2_ref, bta2_ref,
             wca1_ref, bca1_ref, wca2_ref, bca2_ref,
             bmat_ref, bpmat_ref, pat_ref, pad_ref, rmat_ref, tmat_ref,
             x2_ref, out_ref):
    tsf = ts_ref[...].astype(jnp.float32)
    mk = mk_ref[...].astype(jnp.float32)

    x2 = x2_ref[...]
    m = jnp.maximum(x2[:, :640], x2[:, 640:])
    m = jnp.maximum(
        jnp.maximum(jnp.maximum(m[:, 0:128], m[:, 128:256]),
                    jnp.maximum(m[:, 256:384], m[:, 384:512])),
        m[:, 512:640])
    cmax = jnp.maximum(m[:, :64], m[:, 64:128])

    r10 = x2.reshape(_BN, 10, 128)
    r_ev = jnp.max(r10[:, :, :64], axis=2)
    r_od = jnp.max(r10[:, :, 64:], axis=2)
    rowmax2 = jnp.concatenate([r_ev, r_od], axis=1)

    tsx = _dot(tsf, bmat_ref[...])
    onx = _dot(mk, bmat_ref[...])
    rmx = _dot(rowmax2, bpmat_ref[...])
    sel = (tsx == pat_ref[...]) & (onx > 0.0)
    term = jnp.where(sel, rmx, pad_ref[...])
    t_ = term
    for half in (256, 128, 64, 32, 16):
        t_ = jnp.maximum(t_[:, :half], t_[:, half:2 * half])
    tmax = t_[:, :_S]

    ta_h = jnp.maximum(_dot(tmax, wta1_ref[...]) + bta1_ref[...], 0.0)
    ta_w = _dot(ta_h, wta2_ref[...]) + bta2_ref[...]
    ca_h = jnp.maximum(_dot(cmax, wca1_ref[...]) + bca1_ref[...], 0.0)
    ca_w = _dot(ca_h, wca2_ref[...]) + bca2_ref[...]

    ta_g = jnp.where(tsf == 0.0, ta_w[:, 0:1], 0.0)
    for t in range(1, _S):
        ta_g = jnp.where(tsf == float(t), ta_w[:, t:t + 1], ta_g)

    ta_gx = _dot(ta_g, rmat_ref[...])
    mkx = _dot(mk, rmat_ref[...])
    ca_wx = _dot(ca_w, tmat_ref[...])
    z = ta_gx * ca_wx
    gate = 0.5 + 0.5 * jnp.tanh(0.5 * z)
    out_ref[...] = x2 * gate * mkx


@functools.partial(jax.jit, static_argnames=())
def _tca_tc(x, ts, mask, wta1t, bta1, wta2t, bta2, wca1t, bca1, wca2t, bca2):
    x2 = x.reshape(_N, _P * _C)
    mk8 = mask.astype(jnp.int32)
    grid = ((_N + _BN - 1) // _BN,)
    blk = lambda shape: pl.BlockSpec(shape, lambda i: (0,) * len(shape))
    consts = (jnp.asarray(_B_np), jnp.asarray(_BP_np), jnp.asarray(_PAT_np),
              jnp.asarray(_PAD_np), jnp.asarray(_R_np), jnp.asarray(_T_np))
    out2 = pl.pallas_call(
        _tc_body,
        grid=grid,
        in_specs=[
            pl.BlockSpec((_BN, _P), lambda i: (i, 0)),
            pl.BlockSpec((_BN, _P), lambda i: (i, 0)),
            blk(wta1t.shape), blk(bta1.shape),
            blk(wta2t.shape), blk(bta2.shape),
            blk(wca1t.shape), blk(bca1.shape),
            blk(wca2t.shape), blk(bca2.shape),
            blk(consts[0].shape), blk(consts[1].shape), blk(consts[2].shape),
            blk(consts[3].shape), blk(consts[4].shape), blk(consts[5].shape),
            pl.BlockSpec((_BN, _P * _C), lambda i: (i, 0)),
        ],
        out_specs=pl.BlockSpec((_BN, _P * _C), lambda i: (i, 0)),
        out_shape=jax.ShapeDtypeStruct((_N, _P * _C), jnp.float32),
        compiler_params=pltpu.CompilerParams(
            dimension_semantics=("parallel",)),
    )(ts, mk8, wta1t, bta1, wta2t, bta2, wca1t, bca1, wca2t, bca2,
      *consts, x2)
    return out2.reshape(_N, _P, _C)


def kernel(x, timestep, mask, W_ta1, b_ta1, W_ta2, b_ta2,
           W_ca1, b_ca1, W_ca2, b_ca2):
    return _tca_tc(
        x, timestep.astype(jnp.int32), mask,
        W_ta1.T, b_ta1.reshape(1, -1),
        W_ta2.T, b_ta2.reshape(1, -1),
        W_ca1.T, b_ca1.reshape(1, -1),
        W_ca2.T, b_ca2.reshape(1, -1),
    )


# speedup vs baseline: 2.6411x; 1.0339x over previous
import functools

import jax
import jax.numpy as jnp
import numpy as np
from jax.experimental import pallas as pl
from jax.experimental.pallas import tpu as pltpu

_N, _P, _C, _S = 30000, 20, 64, 10
_BN = 1024
_G = 512

_B_np = np.zeros((_P, _G), np.float32)
for _p in range(_P):
    _B_np[_p, _p * 16:(_p + 1) * 16] = 1.0
_BP_np = np.zeros((_P, _G), np.float32)
for _i in range(_P):
    _p = 2 * _i if _i < 10 else 2 * (_i - 10) + 1
    _BP_np[_i, _p * 16:(_p + 1) * 16] = 1.0
_PAT_np = np.tile(np.arange(16, dtype=np.float32), _G // 16)[None, :]
_PAD_np = np.where(np.arange(_G) < _P * 16, 0.0, -3.0e38).astype(np.float32)[None, :]
_R_np = np.zeros((_P, _P * _C), np.float32)
for _p in range(_P):
    _R_np[_p, _p * _C:(_p + 1) * _C] = 1.0
_T_np = np.zeros((_C, _P * _C), np.float32)
for _c in range(_C):
    _T_np[_c, _c::_C] = 1.0


def _dot(a, b):
    return jnp.dot(a, b, preferred_element_type=jnp.float32)


def _tc_body(ts_ref, mk_ref, wta1_ref, bta1_ref, wta2_ref, bta2_ref,
             wca1_ref, bca1_ref, wca2_ref, bca2_ref,
             bmat_ref, bpmat_ref, pat_ref, pad_ref, rmat_ref, tmat_ref,
             x2_ref, out_ref):
    tsf = ts_ref[...].astype(jnp.float32)
    mk = mk_ref[...].astype(jnp.float32)

    x2 = x2_ref[...]
    m = jnp.maximum(x2[:, :640], x2[:, 640:])
    m = jnp.maximum(
        jnp.maximum(jnp.maximum(m[:, 0:128], m[:, 128:256]),
                    jnp.maximum(m[:, 256:384], m[:, 384:512])),
        m[:, 512:640])
    cmax = jnp.maximum(m[:, :64], m[:, 64:128])

    r10 = x2.reshape(_BN, 10, 128)
    r_ev = jnp.max(r10[:, :, :64], axis=2)
    r_od = jnp.max(r10[:, :, 64:], axis=2)
    rowmax2 = jnp.concatenate([r_ev, r_od], axis=1)

    tsx = _dot(tsf, bmat_ref[...])
    onx = _dot(mk, bmat_ref[...])
    rmx = _dot(rowmax2, bpmat_ref[...])
    sel = (tsx == pat_ref[...]) & (onx > 0.0)
    term = jnp.where(sel, rmx, pad_ref[...])
    t_ = term
    for half in (256, 128, 64, 32, 16):
        t_ = jnp.maximum(t_[:, :half], t_[:, half:2 * half])
    tmax = t_[:, :_S]

    ta_h = jnp.maximum(_dot(tmax, wta1_ref[...]) + bta1_ref[...], 0.0)
    ta_w = _dot(ta_h, wta2_ref[...]) + bta2_ref[...]
    ca_h = jnp.maximum(_dot(cmax, wca1_ref[...]) + bca1_ref[...], 0.0)
    ca_w = _dot(ca_h, wca2_ref[...]) + bca2_ref[...]

    ta_g = jnp.where(tsf == 0.0, ta_w[:, 0:1], 0.0)
    for t in range(1, _S):
        ta_g = jnp.where(tsf == float(t), ta_w[:, t:t + 1], ta_g)

    ta_gx = _dot(ta_g, rmat_ref[...])
    mkx = _dot(mk, rmat_ref[...])
    ca_wx = _dot(ca_w, tmat_ref[...])
    z = ta_gx * ca_wx
    gate = 0.5 + 0.5 * jnp.tanh(0.5 * z)
    out_ref[...] = x2 * gate * mkx


@functools.partial(jax.jit, static_argnames=())
def _tca_tc(x, ts, mask, wta1t, bta1, wta2t, bta2, wca1t, bca1, wca2t, bca2):
    x2 = x.reshape(_N, _P * _C)
    mk8 = mask.astype(jnp.int32)
    grid = ((_N + _BN - 1) // _BN,)
    blk = lambda shape: pl.BlockSpec(shape, lambda i: (0,) * len(shape))
    consts = (jnp.asarray(_B_np), jnp.asarray(_BP_np), jnp.asarray(_PAT_np),
              jnp.asarray(_PAD_np), jnp.asarray(_R_np), jnp.asarray(_T_np))
    out2 = pl.pallas_call(
        _tc_body,
        grid=grid,
        in_specs=[
            pl.BlockSpec((_BN, _P), lambda i: (i, 0)),
            pl.BlockSpec((_BN, _P), lambda i: (i, 0)),
            blk(wta1t.shape), blk(bta1.shape),
            blk(wta2t.shape), blk(bta2.shape),
            blk(wca1t.shape), blk(bca1.shape),
            blk(wca2t.shape), blk(bca2.shape),
            blk(consts[0].shape), blk(consts[1].shape), blk(consts[2].shape),
            blk(consts[3].shape), blk(consts[4].shape), blk(consts[5].shape),
            pl.BlockSpec((_BN, _P * _C), lambda i: (i, 0)),
        ],
        out_specs=pl.BlockSpec((_BN, _P * _C), lambda i: (i, 0)),
        out_shape=jax.ShapeDtypeStruct((_N, _P * _C), jnp.float32),
        compiler_params=pltpu.CompilerParams(
            dimension_semantics=("arbitrary",)),
    )(ts, mk8, wta1t, bta1, wta2t, bta2, wca1t, bca1, wca2t, bca2,
      *consts, x2)
    return out2.reshape(_N, _P, _C)


def kernel(x, timestep, mask, W_ta1, b_ta1, W_ta2, b_ta2,
           W_ca1, b_ca1, W_ca2, b_ca2):
    return _tca_tc(
        x, timestep.astype(jnp.int32), mask,
        W_ta1.T, b_ta1.reshape(1, -1),
        W_ta2.T, b_ta2.reshape(1, -1),
        W_ca1.T, b_ca1.reshape(1, -1),
        W_ca2.T, b_ca2.reshape(1, -1),
    )


# speedup vs baseline: 2.6842x; 1.0163x over previous
import functools

import jax
import jax.numpy as jnp
import numpy as np
from jax.experimental import pallas as pl
from jax.experimental.pallas import tpu as pltpu

_N, _P, _C, _S = 30000, 20, 64, 10
_BN = 1200
_G = 512

_B_np = np.zeros((_P, _G), np.float32)
for _p in range(_P):
    _B_np[_p, _p * 16:(_p + 1) * 16] = 1.0
_BP_np = np.zeros((_P, _G), np.float32)
for _i in range(_P):
    _p = 2 * _i if _i < 10 else 2 * (_i - 10) + 1
    _BP_np[_i, _p * 16:(_p + 1) * 16] = 1.0
_PAT_np = np.tile(np.arange(16, dtype=np.float32), _G // 16)[None, :]
_PAD_np = np.where(np.arange(_G) < _P * 16, 0.0, -3.0e38).astype(np.float32)[None, :]
_R_np = np.zeros((_P, _P * _C), np.float32)
for _p in range(_P):
    _R_np[_p, _p * _C:(_p + 1) * _C] = 1.0
_T_np = np.zeros((_C, _P * _C), np.float32)
for _c in range(_C):
    _T_np[_c, _c::_C] = 1.0


def _dot(a, b):
    return jnp.dot(a, b, preferred_element_type=jnp.float32)


def _tc_body(ts_ref, mk_ref, wta1_ref, bta1_ref, wta2_ref, bta2_ref,
             wca1_ref, bca1_ref, wca2_ref, bca2_ref,
             bmat_ref, bpmat_ref, pat_ref, pad_ref, rmat_ref, tmat_ref,
             x2_ref, out_ref):
    tsf = ts_ref[...].astype(jnp.float32)
    mk = mk_ref[...].astype(jnp.float32)

    x2 = x2_ref[...]
    m = jnp.maximum(x2[:, :640], x2[:, 640:])
    m = jnp.maximum(
        jnp.maximum(jnp.maximum(m[:, 0:128], m[:, 128:256]),
                    jnp.maximum(m[:, 256:384], m[:, 384:512])),
        m[:, 512:640])
    cmax = jnp.maximum(m[:, :64], m[:, 64:128])

    r10 = x2.reshape(_BN, 10, 128)
    r_ev = jnp.max(r10[:, :, :64], axis=2)
    r_od = jnp.max(r10[:, :, 64:], axis=2)
    rowmax2 = jnp.concatenate([r_ev, r_od], axis=1)

    tsx = _dot(tsf, bmat_ref[...])
    onx = _dot(mk, bmat_ref[...])
    rmx = _dot(rowmax2, bpmat_ref[...])
    sel = (tsx == pat_ref[...]) & (onx > 0.0)
    term = jnp.where(sel, rmx, pad_ref[...])
    t_ = term
    for half in (256, 128, 64, 32, 16):
        t_ = jnp.maximum(t_[:, :half], t_[:, half:2 * half])
    tmax = t_[:, :_S]

    ta_h = jnp.maximum(_dot(tmax, wta1_ref[...]) + bta1_ref[...], 0.0)
    ta_w = _dot(ta_h, wta2_ref[...]) + bta2_ref[...]
    ca_h = jnp.maximum(_dot(cmax, wca1_ref[...]) + bca1_ref[...], 0.0)
    ca_w = _dot(ca_h, wca2_ref[...]) + bca2_ref[...]

    ta_g = jnp.where(tsf == 0.0, ta_w[:, 0:1], 0.0)
    for t in range(1, _S):
        ta_g = jnp.where(tsf == float(t), ta_w[:, t:t + 1], ta_g)

    ta_gx = _dot(ta_g, rmat_ref[...])
    mkx = _dot(mk, rmat_ref[...])
    ca_wx = _dot(ca_w, tmat_ref[...])
    z = ta_gx * ca_wx
    gate = 0.5 + 0.5 * jnp.tanh(0.5 * z)
    out_ref[...] = x2 * gate * mkx


@functools.partial(jax.jit, static_argnames=())
def _tca_tc(x, ts, mask, wta1t, bta1, wta2t, bta2, wca1t, bca1, wca2t, bca2):
    x2 = x.reshape(_N, _P * _C)
    mk8 = mask.astype(jnp.int32)
    grid = ((_N + _BN - 1) // _BN,)
    blk = lambda shape: pl.BlockSpec(shape, lambda i: (0,) * len(shape))
    consts = (jnp.asarray(_B_np), jnp.asarray(_BP_np), jnp.asarray(_PAT_np),
              jnp.asarray(_PAD_np), jnp.asarray(_R_np), jnp.asarray(_T_np))
    out2 = pl.pallas_call(
        _tc_body,
        grid=grid,
        in_specs=[
            pl.BlockSpec((_BN, _P), lambda i: (i, 0)),
            pl.BlockSpec((_BN, _P), lambda i: (i, 0)),
            blk(wta1t.shape), blk(bta1.shape),
            blk(wta2t.shape), blk(bta2.shape),
            blk(wca1t.shape), blk(bca1.shape),
            blk(wca2t.shape), blk(bca2.shape),
            blk(consts[0].shape), blk(consts[1].shape), blk(consts[2].shape),
            blk(consts[3].shape), blk(consts[4].shape), blk(consts[5].shape),
            pl.BlockSpec((_BN, _P * _C), lambda i: (i, 0)),
        ],
        out_specs=pl.BlockSpec((_BN, _P * _C), lambda i: (i, 0)),
        out_shape=jax.ShapeDtypeStruct((_N, _P * _C), jnp.float32),
        compiler_params=pltpu.CompilerParams(
            dimension_semantics=("arbitrary",)),
    )(ts, mk8, wta1t, bta1, wta2t, bta2, wca1t, bca1, wca2t, bca2,
      *consts, x2)
    return out2.reshape(_N, _P, _C)


def kernel(x, timestep, mask, W_ta1, b_ta1, W_ta2, b_ta2,
           W_ca1, b_ca1, W_ca2, b_ca2):
    return _tca_tc(
        x, timestep.astype(jnp.int32), mask,
        W_ta1.T, b_ta1.reshape(1, -1),
        W_ta2.T, b_ta2.reshape(1, -1),
        W_ca1.T, b_ca1.reshape(1, -1),
        W_ca2.T, b_ca2.reshape(1, -1),
    )
